# baseline (speedup 1.0000x reference)
import functools

import jax
import jax.numpy as jnp
from jax import lax
from jax.experimental import pallas as pl
from jax.experimental.pallas import tpu as pltpu
from jax.experimental.pallas import tpu_sc as plsc

DIM_ = 128
NPOS_ = 100000
HALF_PI = 1.5707963267948966

TR_ = 1000
NC_, NS_ = 2, 16
NW_ = NC_ * NS_
N0_ = 16384
N1_ = 50
B_ = N0_ * N1_
BPW_ = B_ // NW_
CH_ = 400
NCH_ = BPW_ // CH_


_S3, _S5, _S7 = -1.6666654611e-1, 8.3321608736e-3, -1.9515295891e-4
_C2, _C4, _C6 = -0.5, 4.166664568298827e-2, -1.388731625493765e-3
_TWO_OVER_PI = 0.6366197723675814


def _fast_sin(y):
    qf = jnp.floor(y * _TWO_OVER_PI + 0.5)
    r = y - qf * HALF_PI
    qi = qf.astype(jnp.int32)
    u = r * r
    sp = r + r * (u * ((_S7 * u + _S5) * u + _S3))
    cp = 1.0 + u * ((_C6 * u + _C4) * u + _C2)
    res = jnp.where((qi & 1) == 1, cp, sp)
    return jnp.where((qi & 2) == 2, -res, res)


def _table_kernel(out_ref):
    pid = pl.program_id(0)
    row = lax.broadcasted_iota(jnp.int32, (TR_, 1), 0) + pid * TR_
    b = row.astype(jnp.float32) * (1.0 / 10000.0)
    zero_row = row == 0
    logb = jnp.log(jnp.where(zero_row, 1.0, b))

    lane = lax.broadcasted_iota(jnp.int32, (1, DIM_), 1)
    e = (lane // 2).astype(jnp.float32) * (1.0 / DIM_)
    phase = jnp.where(lane % 2 == 1, HALF_PI, 0.0)

    ang = jnp.exp(logb * e)
    ang = jnp.where(zero_row & (lane >= 2), 0.0, ang)
    out_ref[:] = _fast_sin(ang + phase)


def _build_table():
    return pl.pallas_call(
        _table_kernel,
        grid=(NPOS_ // TR_,),
        out_specs=pl.BlockSpec((TR_, DIM_), lambda i: (i, 0)),
        out_shape=jax.ShapeDtypeStruct((NPOS_, DIM_), jnp.float32),
        compiler_params=pltpu.CompilerParams(
            dimension_semantics=("parallel",),
        ),
    )()


@functools.partial(
    pl.kernel,
    out_type=jax.ShapeDtypeStruct((B_, DIM_), jnp.float32),
    mesh=plsc.VectorSubcoreMesh(
        core_axis_name="c", subcore_axis_name="s",
        num_cores=NC_, num_subcores=NS_,
    ),
    scratch_types=[
        pltpu.VMEM((CH_,), jnp.int32),
        pltpu.VMEM((CH_,), jnp.int32),
        pltpu.VMEM((CH_, DIM_), jnp.float32),
        pltpu.VMEM((CH_, DIM_), jnp.float32),
        pltpu.SemaphoreType.DMA,
        pltpu.SemaphoreType.DMA,
        pltpu.SemaphoreType.DMA,
        pltpu.SemaphoreType.DMA,
        pltpu.SemaphoreType.DMA,
        pltpu.SemaphoreType.DMA,
    ],
    compiler_params=pltpu.CompilerParams(use_tc_tiling_on_sc=True),
)
def _sc_gather(table_hbm, idx_hbm, out_hbm,
               idx_v0, idx_v1, rows_v0, rows_v1,
               isem0, isem1, gsem0, gsem1, wsem0, wsem1):
    wid = lax.axis_index("s") * NC_ + lax.axis_index("c")
    base = wid * BPW_
    idx_bufs = (idx_v0, idx_v1)
    row_bufs = (rows_v0, rows_v1)
    isems = (isem0, isem1)
    gsems = (gsem0, gsem1)
    wsems = (wsem0, wsem1)

    def start_idx(c, b):
        off = base + c * CH_
        pltpu.async_copy(idx_hbm.at[pl.ds(off, CH_)], idx_bufs[b], isems[b])

    def wait_idx(c, b):
        off = base + c * CH_
        pltpu.make_async_copy(idx_hbm.at[pl.ds(off, CH_)], idx_bufs[b],
                              isems[b]).wait()

    def start_gather(b):
        pltpu.async_copy(table_hbm.at[idx_bufs[b]], row_bufs[b], gsems[b])

    start_idx(0, 0)
    start_idx(1, 1)
    wait_idx(0, 0)
    start_gather(0)
    wait_idx(1, 1)
    start_gather(1)

    def body(it, carry):
        for b in range(2):
            c = 2 * it + b
            off = base + c * CH_
            pltpu.make_async_copy(
                table_hbm.at[idx_bufs[b]], row_bufs[b], gsems[b]).wait()
            pltpu.async_copy(row_bufs[b], out_hbm.at[pl.ds(off, CH_)],
                             wsems[b])

            @pl.when(c + 2 < NCH_)
            def _():
                start_idx(c + 2, b)

            pltpu.make_async_copy(row_bufs[b], out_hbm.at[pl.ds(off, CH_)],
                                  wsems[b]).wait()

            @pl.when(c + 2 < NCH_)
            def _():
                wait_idx(c + 2, b)
                start_gather(b)
        return carry

    lax.fori_loop(0, NCH_ // 2, body, 0)


@jax.jit
def kernel(t):
    table = _build_table()
    idx = t.T.reshape(B_).astype(jnp.int32)
    out = _sc_gather(table, idx)
    return out.reshape(N1_, N0_, DIM_).transpose(1, 0, 2)


# speedup vs baseline: 1.0190x; 1.0190x over previous
import functools

import jax
import jax.numpy as jnp
from jax import lax
from jax.experimental import pallas as pl
from jax.experimental.pallas import tpu as pltpu
from jax.experimental.pallas import tpu_sc as plsc

DIM_ = 128
NPOS_ = 100000
HALF_PI = 1.5707963267948966

TR_ = 2000
NC_, NS_ = 2, 16
NW_ = NC_ * NS_
N0_ = 16384
N1_ = 50
B_ = N0_ * N1_
BPW_ = B_ // NW_
CH_ = 400
NCH_ = BPW_ // CH_


_S3, _S5, _S7 = -1.6666654611e-1, 8.3321608736e-3, -1.9515295891e-4
_C2, _C4, _C6 = -0.5, 4.166664568298827e-2, -1.388731625493765e-3
_TWO_OVER_PI = 0.6366197723675814


def _fast_sin(y):
    qf = jnp.floor(y * _TWO_OVER_PI + 0.5)
    r = y - qf * HALF_PI
    qi = qf.astype(jnp.int32)
    u = r * r
    sp = r + r * (u * ((_S7 * u + _S5) * u + _S3))
    cp = 1.0 + u * ((_C6 * u + _C4) * u + _C2)
    res = jnp.where((qi & 1) == 1, cp, sp)
    return jnp.where((qi & 2) == 2, -res, res)


def _table_kernel(out_ref):
    pid = pl.program_id(0)
    row = lax.broadcasted_iota(jnp.int32, (TR_, 1), 0) + pid * TR_
    b = jnp.maximum(row, 1).astype(jnp.float32) * (1.0 / 10000.0)
    logb = jnp.log(b)

    lane = lax.broadcasted_iota(jnp.int32, (1, DIM_), 1)
    e = (lane // 2).astype(jnp.float32) * (1.0 / DIM_)
    phase = jnp.where(lane % 2 == 1, HALF_PI, 0.0)

    ang = jnp.exp(logb * e)
    out_ref[:] = _fast_sin(ang + phase)

    @pl.when(pid == 0)
    def _():
        row0 = jnp.where(lane == 0, 0.8414709848078965,
                         jnp.where(lane == 1, 0.5403023058681398,
                                   phase / HALF_PI))
        out_ref[pl.ds(0, 1), :] = row0


def _build_table():
    return pl.pallas_call(
        _table_kernel,
        grid=(NPOS_ // TR_,),
        out_specs=pl.BlockSpec((TR_, DIM_), lambda i: (i, 0)),
        out_shape=jax.ShapeDtypeStruct((NPOS_, DIM_), jnp.float32),
        compiler_params=pltpu.CompilerParams(
            dimension_semantics=("parallel",),
        ),
    )()


@functools.partial(
    pl.kernel,
    out_type=jax.ShapeDtypeStruct((B_, DIM_), jnp.float32),
    mesh=plsc.VectorSubcoreMesh(
        core_axis_name="c", subcore_axis_name="s",
        num_cores=NC_, num_subcores=NS_,
    ),
    scratch_types=[
        pltpu.VMEM((CH_,), jnp.int32),
        pltpu.VMEM((CH_,), jnp.int32),
        pltpu.VMEM((CH_, DIM_), jnp.float32),
        pltpu.VMEM((CH_, DIM_), jnp.float32),
        pltpu.SemaphoreType.DMA,
        pltpu.SemaphoreType.DMA,
        pltpu.SemaphoreType.DMA,
        pltpu.SemaphoreType.DMA,
        pltpu.SemaphoreType.DMA,
        pltpu.SemaphoreType.DMA,
    ],
    compiler_params=pltpu.CompilerParams(use_tc_tiling_on_sc=True),
)
def _sc_gather(table_hbm, idx_hbm, out_hbm,
               idx_v0, idx_v1, rows_v0, rows_v1,
               isem0, isem1, gsem0, gsem1, wsem0, wsem1):
    wid = lax.axis_index("s") * NC_ + lax.axis_index("c")
    base = wid * BPW_
    idx_bufs = (idx_v0, idx_v1)
    row_bufs = (rows_v0, rows_v1)
    isems = (isem0, isem1)
    gsems = (gsem0, gsem1)
    wsems = (wsem0, wsem1)

    def start_idx(c, b):
        off = base + c * CH_
        pltpu.async_copy(idx_hbm.at[pl.ds(off, CH_)], idx_bufs[b], isems[b])

    def wait_idx(c, b):
        off = base + c * CH_
        pltpu.make_async_copy(idx_hbm.at[pl.ds(off, CH_)], idx_bufs[b],
                              isems[b]).wait()

    def start_gather(b):
        pltpu.async_copy(table_hbm.at[idx_bufs[b]], row_bufs[b], gsems[b])

    start_idx(0, 0)
    start_idx(1, 1)
    wait_idx(0, 0)
    start_gather(0)
    wait_idx(1, 1)
    start_gather(1)

    def body(it, carry):
        for b in range(2):
            c = 2 * it + b
            off = base + c * CH_
            pltpu.make_async_copy(
                table_hbm.at[idx_bufs[b]], row_bufs[b], gsems[b]).wait()
            pltpu.async_copy(row_bufs[b], out_hbm.at[pl.ds(off, CH_)],
                             wsems[b])

            @pl.when(c + 2 < NCH_)
            def _():
                start_idx(c + 2, b)

            pltpu.make_async_copy(row_bufs[b], out_hbm.at[pl.ds(off, CH_)],
                                  wsems[b]).wait()

            @pl.when(c + 2 < NCH_)
            def _():
                wait_idx(c + 2, b)
                start_gather(b)
        return carry

    lax.fori_loop(0, NCH_ // 2, body, 0)


@jax.jit
def kernel(t):
    table = _build_table()
    idx = t.T.reshape(B_).astype(jnp.int32)
    out = _sc_gather(table, idx)
    return out.reshape(N1_, N0_, DIM_).transpose(1, 0, 2)
